# baseline (speedup 1.0000x reference)
import functools

import jax
import jax.numpy as jnp
from jax import lax
from jax.experimental import pallas as pl
from jax.experimental.pallas import tpu as pltpu
from jax.experimental.pallas import tpu_sc as plsc

F = 64
B = 16384
NC = 2
NS = 16
L = 16
NW = NC * NS
BPW = B // NW
CHUNK = 128
NCH = BPW // CHUNK
W = 2 * F

_mesh = plsc.VectorSubcoreMesh(core_axis_name="c", subcore_axis_name="s")


@functools.partial(
    pl.kernel,
    out_type=jax.ShapeDtypeStruct((B,), jnp.float32),
    mesh=_mesh,
    compiler_params=pltpu.CompilerParams(
        use_tc_tiling_on_sc=True, needs_layout_passes=False),
    scratch_types=[
        pltpu.VMEM((NCH, CHUNK), jnp.int32),
        pltpu.VMEM((NCH, CHUNK), jnp.int32),
        pltpu.VMEM((2, CHUNK, W), jnp.float32),
        pltpu.VMEM((2, CHUNK, W), jnp.float32),
        pltpu.VMEM((BPW * L,), jnp.float32),
        pltpu.VMEM((BPW,), jnp.float32),
        pltpu.SemaphoreType.DMA,
        pltpu.SemaphoreType.DMA,
        pltpu.SemaphoreType.DMA,
        pltpu.SemaphoreType.DMA,
    ],
)
def _sc_dot(users_hbm, items_hbm, utv_hbm, itv_hbm, out_hbm,
            uidx, iidx, urows, irows, psum, outv, su0, su1, si0, si1):
    wid = lax.axis_index("s") * NC + lax.axis_index("c")
    base = wid * BPW
    sems_u = (su0, su1)
    sems_i = (si0, si1)

    for j in range(NCH):
        off = base + j * CHUNK
        pltpu.sync_copy(users_hbm.at[pl.ds(off, CHUNK)], uidx.at[j])
        pltpu.sync_copy(items_hbm.at[pl.ds(off, CHUNK)], iidx.at[j])

    def fire_batch(q):
        s = q % 2

        def fire_group(g, carry):
            uvec = uidx[q, pl.ds(g * L, L)] >> 1
            ivec = iidx[q, pl.ds(g * L, L)] >> 1
            for k in range(L):
                slot = g * L + k
                pltpu.async_copy(
                    utv_hbm.at[pl.ds(uvec[k], 1), :],
                    urows.at[s].at[pl.ds(slot, 1), :], sems_u[s])
                pltpu.async_copy(
                    itv_hbm.at[pl.ds(ivec[k], 1), :],
                    irows.at[s].at[pl.ds(slot, 1), :], sems_i[s])
            return carry

        lax.fori_loop(0, CHUNK // L, fire_group, 0)

    def drain_batch(q):
        s = q % 2
        pltpu.make_async_copy(
            utv_hbm.at[pl.ds(0, CHUNK), :], urows.at[s], sems_u[s]).wait()
        pltpu.make_async_copy(
            itv_hbm.at[pl.ds(0, CHUNK), :], irows.at[s], sems_i[s]).wait()

    def compute_batch(q):
        s = q % 2

        def compute_group(g, carry):
            uvec = uidx[q, pl.ds(g * L, L)]
            ivec = iidx[q, pl.ds(g * L, L)]
            for k in range(L):
                r = g * L + k
                mu = jnp.full((L,), uvec[k] & 1, jnp.int32) == 1
                mi = jnp.full((L,), ivec[k] & 1, jnp.int32) == 1
                acc = None
                for c0 in range(F // L):
                    ulo = urows[s, r, pl.ds(c0 * L, L)]
                    uhi = urows[s, r, pl.ds(F + c0 * L, L)]
                    ilo = irows[s, r, pl.ds(c0 * L, L)]
                    ihi = irows[s, r, pl.ds(F + c0 * L, L)]
                    u = jnp.where(mu, uhi, ulo)
                    v = jnp.where(mi, ihi, ilo)
                    acc = u * v if acc is None else acc + u * v
                psum[pl.ds((q * CHUNK + r) * L, L)] = acc
            return carry

        lax.fori_loop(0, CHUNK // L, compute_group, 0)

    fire_batch(0)
    for q in range(NCH):
        if q + 1 < NCH:
            fire_batch(q + 1)
        drain_batch(q)
        compute_batch(q)

    lanes = lax.iota(jnp.int32, L) * L

    def red_body(g, carry):
        bi = lanes + g * (L * L)
        acc = plsc.load_gather(psum, [bi])
        for p in range(1, L):
            acc = acc + plsc.load_gather(psum, [bi + p])
        outv[pl.ds(g * L, L)] = acc
        return carry

    lax.fori_loop(0, BPW // L, red_body, 0)

    pltpu.sync_copy(outv, out_hbm.at[pl.ds(base, BPW)])


def kernel(users, items, user_table, item_table):
    users = users.astype(jnp.int32)
    items = items.astype(jnp.int32)
    utv = user_table.reshape(user_table.shape[0] // 2, W)
    itv = item_table.reshape(item_table.shape[0] // 2, W)
    return _sc_dot(users, items, utv, itv)


# speedup vs baseline: 2.4167x; 2.4167x over previous
import functools

import jax
import jax.numpy as jnp
from jax import lax
from jax.experimental import pallas as pl
from jax.experimental.pallas import tpu as pltpu
from jax.experimental.pallas import tpu_sc as plsc

F = 64
B = 16384
NC = 2
NS = 16
L = 16
NW = NC * NS
BPW = B // NW
CHUNK = 128
NCH = BPW // CHUNK
NG = BPW // L

_mesh = plsc.VectorSubcoreMesh(core_axis_name="c", subcore_axis_name="s")


@functools.partial(
    pl.kernel,
    out_type=jax.ShapeDtypeStruct((B,), jnp.float32),
    mesh=_mesh,
    compiler_params=pltpu.CompilerParams(
        use_tc_tiling_on_sc=True, needs_layout_passes=False),
    scratch_types=[
        pltpu.VMEM((NCH, CHUNK), jnp.int32),
        pltpu.VMEM((NCH, CHUNK), jnp.int32),
        pltpu.VMEM((2, CHUNK, 1, F), jnp.float32),
        pltpu.VMEM((2, CHUNK, 1, F), jnp.float32),
        pltpu.VMEM((BPW * L,), jnp.float32),
        pltpu.VMEM((BPW,), jnp.float32),
        pltpu.SemaphoreType.DMA,
        pltpu.SemaphoreType.DMA,
        pltpu.SemaphoreType.DMA,
        pltpu.SemaphoreType.DMA,
    ],
)
def _sc_dot(users_hbm, items_hbm, ut_hbm, it_hbm, out_hbm,
            uidx, iidx, urows, irows, psum, outv, su0, su1, si0, si1):
    wid = lax.axis_index("s") * NC + lax.axis_index("c")
    base = wid * BPW
    sems_u = (su0, su1)
    sems_i = (si0, si1)

    for j in range(NCH):
        off = base + j * CHUNK
        pltpu.sync_copy(users_hbm.at[pl.ds(off, CHUNK)], uidx.at[j])
        pltpu.sync_copy(items_hbm.at[pl.ds(off, CHUNK)], iidx.at[j])

    def fire_batch(q):
        s = q % 2

        def fire_group(g, carry):
            uvec = uidx[q, pl.ds(g * L, L)]
            ivec = iidx[q, pl.ds(g * L, L)]
            for k in range(L):
                slot = g * L + k
                pltpu.async_copy(
                    ut_hbm.at[pl.ds(uvec[k] >> 3, 1), pl.ds(uvec[k] & 7, 1), :],
                    urows.at[s].at[pl.ds(slot, 1)], sems_u[s])
                pltpu.async_copy(
                    it_hbm.at[pl.ds(ivec[k] >> 3, 1), pl.ds(ivec[k] & 7, 1), :],
                    irows.at[s].at[pl.ds(slot, 1)], sems_i[s])
            return carry

        lax.fori_loop(0, CHUNK // L, fire_group, 0)

    def drain_batch(q):
        s = q % 2
        pltpu.make_async_copy(
            ut_hbm.at[pl.ds(0, CHUNK), pl.ds(0, 1), :],
            urows.at[s], sems_u[s]).wait()
        pltpu.make_async_copy(
            it_hbm.at[pl.ds(0, CHUNK), pl.ds(0, 1), :],
            irows.at[s], sems_i[s]).wait()

    def compute_batch(q):
        s = q % 2

        def compute_group(g, carry):
            for k in range(L):
                r = g * L + k
                acc = None
                for c0 in range(F // L):
                    u = urows[s, r, 0, pl.ds(c0 * L, L)]
                    v = irows[s, r, 0, pl.ds(c0 * L, L)]
                    acc = u * v if acc is None else acc + u * v
                psum[pl.ds((q * CHUNK + r) * L, L)] = acc
            return carry

        lax.fori_loop(0, CHUNK // L, compute_group, 0)

    fire_batch(0)
    for q in range(NCH):
        if q + 1 < NCH:
            fire_batch(q + 1)
        drain_batch(q)
        compute_batch(q)

    lanes = lax.iota(jnp.int32, L) * L

    def red_body(g, carry):
        bi = lanes + g * (L * L)
        acc = plsc.load_gather(psum, [bi])
        for p in range(1, L):
            acc = acc + plsc.load_gather(psum, [bi + p])
        outv[pl.ds(g * L, L)] = acc
        return carry

    lax.fori_loop(0, NG, red_body, 0)

    pltpu.sync_copy(outv, out_hbm.at[pl.ds(base, BPW)])


def kernel(users, items, user_table, item_table):
    ut3 = user_table.reshape(user_table.shape[0] // 8, 8, F)
    it3 = item_table.reshape(item_table.shape[0] // 8, 8, F)
    return _sc_dot(users.astype(jnp.int32), items.astype(jnp.int32),
                   ut3, it3)


# speedup vs baseline: 2.4502x; 1.0139x over previous
import functools

import jax
import jax.numpy as jnp
from jax import lax
from jax.experimental import pallas as pl
from jax.experimental.pallas import tpu as pltpu
from jax.experimental.pallas import tpu_sc as plsc

F = 64
B = 16384
NC = 2
NS = 16
L = 16
NW = NC * NS
BPW = B // NW
CHUNK = 128
NCH = BPW // CHUNK
NG = BPW // L

_mesh = plsc.VectorSubcoreMesh(core_axis_name="c", subcore_axis_name="s")


@functools.partial(
    pl.kernel,
    out_type=jax.ShapeDtypeStruct((B,), jnp.float32),
    mesh=_mesh,
    compiler_params=pltpu.CompilerParams(
        use_tc_tiling_on_sc=True, needs_layout_passes=False),
    scratch_types=[
        pltpu.VMEM((BPW,), jnp.int32),
        pltpu.VMEM((BPW,), jnp.int32),
        pltpu.VMEM((2, CHUNK, 1, F), jnp.float32),
        pltpu.VMEM((2, CHUNK, 1, F), jnp.float32),
        pltpu.VMEM((BPW * L,), jnp.float32),
        pltpu.VMEM((BPW,), jnp.float32),
        pltpu.SemaphoreType.DMA,
        pltpu.SemaphoreType.DMA,
        pltpu.SemaphoreType.DMA,
        pltpu.SemaphoreType.DMA,
    ],
)
def _sc_dot(users_hbm, items_hbm, ut_hbm, it_hbm, out_hbm,
            uidx, iidx, urows, irows, psum, outv, su0, su1, si0, si1):
    wid = lax.axis_index("s") * NC + lax.axis_index("c")
    base = wid * BPW
    sems_u = (su0, su1)
    sems_i = (si0, si1)

    ci = pltpu.async_copy(users_hbm.at[pl.ds(base, BPW)], uidx, su0)
    cj = pltpu.async_copy(items_hbm.at[pl.ds(base, BPW)], iidx, si0)
    ci.wait()
    cj.wait()

    def fire_batch(q):
        s = q % 2

        def fire_group(g, carry):
            uvec = uidx[pl.ds(q * CHUNK + g * L, L)]
            ivec = iidx[pl.ds(q * CHUNK + g * L, L)]
            for k in range(L):
                slot = g * L + k
                pltpu.async_copy(
                    ut_hbm.at[pl.ds(uvec[k] >> 3, 1), pl.ds(uvec[k] & 7, 1), :],
                    urows.at[s].at[pl.ds(slot, 1)], sems_u[s])
                pltpu.async_copy(
                    it_hbm.at[pl.ds(ivec[k] >> 3, 1), pl.ds(ivec[k] & 7, 1), :],
                    irows.at[s].at[pl.ds(slot, 1)], sems_i[s])
            return carry

        lax.fori_loop(0, CHUNK // L, fire_group, 0)

    def drain_batch(q):
        s = q % 2
        pltpu.make_async_copy(
            ut_hbm.at[pl.ds(0, CHUNK), pl.ds(0, 1), :],
            urows.at[s], sems_u[s]).wait()
        pltpu.make_async_copy(
            it_hbm.at[pl.ds(0, CHUNK), pl.ds(0, 1), :],
            irows.at[s], sems_i[s]).wait()

    def compute_batch(q):
        s = q % 2

        def compute_group(g, carry):
            for k in range(L):
                r = g * L + k
                acc = None
                for c0 in range(F // L):
                    u = urows[s, r, 0, pl.ds(c0 * L, L)]
                    v = irows[s, r, 0, pl.ds(c0 * L, L)]
                    acc = u * v if acc is None else acc + u * v
                psum[pl.ds((q * CHUNK + r) * L, L)] = acc
            return carry

        lax.fori_loop(0, CHUNK // L, compute_group, 0)

    fire_batch(0)
    for q in range(NCH):
        if q + 1 < NCH:
            fire_batch(q + 1)
        drain_batch(q)
        compute_batch(q)

    lanes = lax.iota(jnp.int32, L) * L

    def red_body(g, carry):
        bi = lanes + g * (L * L)
        acc = plsc.load_gather(psum, [bi])
        for p in range(1, L):
            acc = acc + plsc.load_gather(psum, [bi + p])
        outv[pl.ds(g * L, L)] = acc
        return carry

    lax.fori_loop(0, NG, red_body, 0)

    pltpu.sync_copy(outv, out_hbm.at[pl.ds(base, BPW)])


def kernel(users, items, user_table, item_table):
    ut3 = user_table.reshape(user_table.shape[0] // 8, 8, F)
    it3 = item_table.reshape(item_table.shape[0] // 8, 8, F)
    return _sc_dot(users.astype(jnp.int32), items.astype(jnp.int32),
                   ut3, it3)


# speedup vs baseline: 2.5356x; 1.0348x over previous
import functools

import jax
import jax.numpy as jnp
from jax import lax
from jax.experimental import pallas as pl
from jax.experimental.pallas import tpu as pltpu
from jax.experimental.pallas import tpu_sc as plsc

F = 64
B = 16384
NC = 2
NS = 16
L = 16
NW = NC * NS
BPW = B // NW
CHUNK = 128
NCH = BPW // CHUNK
NG = BPW // L

_mesh = plsc.VectorSubcoreMesh(core_axis_name="c", subcore_axis_name="s")


@functools.partial(
    pl.kernel,
    out_type=jax.ShapeDtypeStruct((B,), jnp.float32),
    mesh=_mesh,
    compiler_params=pltpu.CompilerParams(
        use_tc_tiling_on_sc=True, needs_layout_passes=False),
    scratch_types=[
        pltpu.VMEM((BPW,), jnp.int32),
        pltpu.VMEM((BPW,), jnp.int32),
        pltpu.VMEM((2, CHUNK, 1, F), jnp.float32),
        pltpu.VMEM((2, CHUNK, F), jnp.float32),
        pltpu.VMEM((BPW * L,), jnp.float32),
        pltpu.VMEM((BPW,), jnp.float32),
        pltpu.SemaphoreType.DMA,
        pltpu.SemaphoreType.DMA,
        pltpu.SemaphoreType.DMA,
        pltpu.SemaphoreType.DMA,
    ],
)
def _sc_dot(users_hbm, items_hbm, ut_hbm, it_hbm, out_hbm,
            uidx, iidx, urows, irows, psum, outv, su0, su1, si0, si1):
    wid = lax.axis_index("s") * NC + lax.axis_index("c")
    base = wid * BPW
    sems_u = (su0, su1)
    sems_i = (si0, si1)

    ci = pltpu.async_copy(users_hbm.at[pl.ds(base, BPW)], uidx, su0)
    cj = pltpu.async_copy(items_hbm.at[pl.ds(base, BPW)], iidx, si0)
    ci.wait()
    cj.wait()

    def fire_batch(q):
        s = q % 2

        def fire_group(g, carry):
            uvec = uidx[pl.ds(q * CHUNK + g * L, L)]
            ivec = iidx[pl.ds(q * CHUNK + g * L, L)]
            for k in range(L):
                slot = g * L + k
                pltpu.async_copy(
                    ut_hbm.at[pl.ds(uvec[k] >> 3, 1), pl.ds(uvec[k] & 7, 1), :],
                    urows.at[s].at[pl.ds(slot, 1)], sems_u[s])
                pltpu.async_copy(
                    it_hbm.at[pl.ds(ivec[k], 1), :],
                    irows.at[s].at[pl.ds(slot, 1), :], sems_i[s])
            return carry

        lax.fori_loop(0, CHUNK // L, fire_group, 0)

    def drain_batch(q):
        s = q % 2
        pltpu.make_async_copy(
            ut_hbm.at[pl.ds(0, CHUNK), pl.ds(0, 1), :],
            urows.at[s], sems_u[s]).wait()
        pltpu.make_async_copy(
            it_hbm.at[pl.ds(0, CHUNK), :], irows.at[s], sems_i[s]).wait()

    def compute_batch(q):
        s = q % 2

        def compute_group(g, carry):
            for k in range(L):
                r = g * L + k
                acc = None
                for c0 in range(F // L):
                    u = urows[s, r, 0, pl.ds(c0 * L, L)]
                    v = irows[s, r, pl.ds(c0 * L, L)]
                    acc = u * v if acc is None else acc + u * v
                psum[pl.ds((q * CHUNK + r) * L, L)] = acc
            return carry

        lax.fori_loop(0, CHUNK // L, compute_group, 0)

    fire_batch(0)
    for q in range(NCH):
        if q + 1 < NCH:
            fire_batch(q + 1)
        drain_batch(q)
        compute_batch(q)

    lanes = lax.iota(jnp.int32, L) * L

    def red_body(g, carry):
        bi = lanes + g * (L * L)
        acc = plsc.load_gather(psum, [bi])
        for p in range(1, L):
            acc = acc + plsc.load_gather(psum, [bi + p])
        outv[pl.ds(g * L, L)] = acc
        return carry

    lax.fori_loop(0, NG, red_body, 0)

    pltpu.sync_copy(outv, out_hbm.at[pl.ds(base, BPW)])


def kernel(users, items, user_table, item_table):
    ut3 = user_table.reshape(user_table.shape[0] // 8, 8, F)
    return _sc_dot(users.astype(jnp.int32), items.astype(jnp.int32),
                   ut3, item_table)


# speedup vs baseline: 2.5401x; 1.0018x over previous
import functools

import jax
import jax.numpy as jnp
from jax import lax
from jax.experimental import pallas as pl
from jax.experimental.pallas import tpu as pltpu
from jax.experimental.pallas import tpu_sc as plsc

F = 64
B = 16384
NC = 2
NS = 16
L = 16
NW = NC * NS
BPW = B // NW
CHUNK = 128
NCH = BPW // CHUNK
NG = BPW // L

_mesh = plsc.VectorSubcoreMesh(core_axis_name="c", subcore_axis_name="s")


@functools.partial(
    pl.kernel,
    out_type=jax.ShapeDtypeStruct((B,), jnp.float32),
    mesh=_mesh,
    compiler_params=pltpu.CompilerParams(
        use_tc_tiling_on_sc=True, needs_layout_passes=False),
    scratch_types=[
        pltpu.VMEM((BPW,), jnp.int32),
        pltpu.VMEM((BPW,), jnp.int32),
        pltpu.VMEM((2, CHUNK, 1, F), jnp.float32),
        pltpu.VMEM((2, CHUNK, F), jnp.float32),
        pltpu.VMEM((BPW * L,), jnp.float32),
        pltpu.VMEM((BPW,), jnp.float32),
        pltpu.SemaphoreType.DMA,
        pltpu.SemaphoreType.DMA,
        pltpu.SemaphoreType.DMA,
        pltpu.SemaphoreType.DMA,
    ],
)
def _sc_dot(users_hbm, items_hbm, ut_hbm, it_hbm, out_hbm,
            uidx, iidx, urows, irows, psum, outv, su0, su1, si0, si1):
    wid = lax.axis_index("s") * NC + lax.axis_index("c")
    base = wid * BPW
    sems_u = (su0, su1)
    sems_i = (si0, si1)

    ci = pltpu.async_copy(users_hbm.at[pl.ds(base, BPW)], uidx, su0)
    cj = pltpu.async_copy(items_hbm.at[pl.ds(base, BPW)], iidx, si0)
    ci.wait()
    cj.wait()

    def fire_batch(q):
        s = q % 2

        def fire_group(g, carry):
            uvec = uidx[pl.ds(q * CHUNK + g * L, L)]
            ivec = iidx[pl.ds(q * CHUNK + g * L, L)]
            uq = uvec >> 3
            ur = uvec & 7
            for k in range(L):
                slot = g * L + k
                pltpu.async_copy(
                    ut_hbm.at[pl.ds(uq[k], 1), pl.ds(ur[k], 1), :],
                    urows.at[s].at[pl.ds(slot, 1)], sems_u[s])
                pltpu.async_copy(
                    it_hbm.at[pl.ds(ivec[k], 1), :],
                    irows.at[s].at[pl.ds(slot, 1), :], sems_i[s])
            return carry

        lax.fori_loop(0, CHUNK // L, fire_group, 0)

    def drain_batch(q):
        s = q % 2
        pltpu.make_async_copy(
            ut_hbm.at[pl.ds(0, CHUNK), pl.ds(0, 1), :],
            urows.at[s], sems_u[s]).wait()
        pltpu.make_async_copy(
            it_hbm.at[pl.ds(0, CHUNK), :], irows.at[s], sems_i[s]).wait()

    def compute_batch(q):
        s = q % 2

        def compute_group(g, carry):
            for k in range(L):
                r = g * L + k
                acc = None
                for c0 in range(F // L):
                    u = urows[s, r, 0, pl.ds(c0 * L, L)]
                    v = irows[s, r, pl.ds(c0 * L, L)]
                    acc = u * v if acc is None else acc + u * v
                psum[pl.ds((q * CHUNK + r) * L, L)] = acc
            return carry

        lax.fori_loop(0, CHUNK // L, compute_group, 0)

    fire_batch(0)
    for q in range(NCH):
        if q + 1 < NCH:
            fire_batch(q + 1)
        drain_batch(q)
        compute_batch(q)

    lanes = lax.iota(jnp.int32, L) * L

    def red_body(g, carry):
        bi = lanes + g * (L * L)
        acc = plsc.load_gather(psum, [bi])
        for p in range(1, L):
            acc = acc + plsc.load_gather(psum, [bi + p])
        outv[pl.ds(g * L, L)] = acc
        return carry

    lax.fori_loop(0, NG, red_body, 0)

    pltpu.sync_copy(outv, out_hbm.at[pl.ds(base, BPW)])


def kernel(users, items, user_table, item_table):
    ut3 = user_table.reshape(user_table.shape[0] // 8, 8, F)
    return _sc_dot(users.astype(jnp.int32), items.astype(jnp.int32),
                   ut3, item_table)
